# baseline (speedup 1.0000x reference)
import functools

import jax
import jax.numpy as jnp
from jax.experimental import pallas as pl
from jax.experimental.pallas import tpu as pltpu

L, B, D, H = 1024, 8, 256, 256
CH = 64


def _gates(gx, gh, h_prev):
    r = jax.nn.sigmoid(gx[:, :H] + gh[:, :H])
    z = jax.nn.sigmoid(gx[:, H:2 * H] + gh[:, H:2 * H])
    n = jnp.tanh(gx[:, 2 * H:] + r * gh[:, 2 * H:])
    return (1.0 - z) * n + z * h_prev


def _smask(conds):
    cols = [jnp.broadcast_to(jnp.where(c, 1.0, 0.0), (1, 1)) for c in conds]
    return jnp.concatenate(cols, axis=0)


def _any(conds):
    return functools.reduce(jnp.logical_or, conds)


def _bf(v):
    return v.astype(jnp.bfloat16)


def _rnn_kernel(heads_ref, x_ref, wdt_ref, udt_ref, bdt_ref, wtd_ref,
                utd_ref, btd_ref, out_ref, child_sum_ref, htd_ref,
                gxdt_ref, gxtd_ref, cdt_ref, ctd_ref):
    child_sum_ref[...] = jnp.zeros((L, B, H), jnp.float32)

    def pair(c, s2, carry):
        (hs0, hs1s), (hp0, hp1s) = carry
        t = c * CH + 2 * s2
        i = L - 1 - t

        hd0 = [heads_ref[b, i] for b in range(B)]
        hd1 = [heads_ref[b, i - 1] for b in range(B)]
        ht1 = [heads_ref[b, t + 1] for b in range(B)]
        tn2 = jnp.minimum(t + 2, L - 1)
        tn3 = jnp.minimum(t + 3, L - 1)
        ht2 = [heads_ref[b, tn2] for b in range(B)]
        ht3 = [heads_ref[b, tn3] for b in range(B)]

        gxd = gxdt_ref[pl.ds(CH - 2 - 2 * s2, 2)]
        gx0 = gxd[1].reshape(B, 3 * H)
        gx1 = gxd[0].reshape(B, 3 * H)
        ghd = jnp.dot(_bf(jnp.concatenate([hs0, hs1s], axis=0)),
                      udt_ref[...], preferred_element_type=jnp.float32)
        h_dt0 = _gates(gx0, ghd[:B], hs0)

        m_dd = _smask([hd0[b] == i - 1 for b in range(B)])
        hs1 = hs1s + h_dt0 * m_dd

        @pl.when(_any([hd0[b] == i - 1 for b in range(B)]))
        def _corr_dt():
            cdt_ref[...] = jnp.dot(_bf(hs1), udt_ref[...],
                                   preferred_element_type=jnp.float32)

        gh1 = jnp.where(m_dd > 0.0, cdt_ref[...], ghd[B:])
        h_dt1 = _gates(gx1, gh1, hs1)
        out_ref[:, pl.ds(i, 1), 0:H] = h_dt0.reshape(B, 1, H)
        out_ref[:, pl.ds(i - 1, 1), 0:H] = h_dt1.reshape(B, 1, H)

        pf = child_sum_ref[pl.ds(jnp.maximum(i - 3, 0), 2)]
        m02 = _smask([hd0[b] == i - 2 for b in range(B)])
        m03 = _smask([hd0[b] == i - 3 for b in range(B)])
        m12 = _smask([hd1[b] == i - 2 for b in range(B)])
        m13 = _smask([hd1[b] == i - 3 for b in range(B)])
        hs_n0 = pf[1].reshape(B, H) + h_dt0 * m02 + h_dt1 * m12
        hs_n1 = pf[0].reshape(B, H) + h_dt0 * m03 + h_dt1 * m13

        g1 = jnp.where(i - 1 > 0, 1.0, 0.0)
        e0 = (1.0 - m_dd) * (1.0 - m02) * (1.0 - m03)
        e1 = (1.0 - m12) * (1.0 - m13) * g1
        upd0 = h_dt0 * e0
        upd1 = h_dt1 * e1
        sib = _smask([hd0[b] == hd1[b] for b in range(B)])
        u0s = upd0 * sib
        u1s = upd1 * sib
        l0 = [child_sum_ref[pl.ds(hd0[b], 1), b, :] for b in range(B)]
        l1 = [child_sum_ref[pl.ds(hd1[b], 1), b, :] for b in range(B)]
        for b in range(B):
            child_sum_ref[pl.ds(hd0[b], 1), b, :] = (
                l0[b] + upd0[b:b + 1, :] + u1s[b:b + 1, :])
        for b in range(B):
            child_sum_ref[pl.ds(hd1[b], 1), b, :] = (
                l1[b] + upd1[b:b + 1, :] + u0s[b:b + 1, :])

        gxt = gxtd_ref[pl.ds(2 * s2, 2)]
        gt0 = gxt[0].reshape(B, 3 * H)
        gt1 = gxt[1].reshape(B, 3 * H)
        ght = jnp.dot(_bf(jnp.concatenate([hp0, hp1s], axis=0)),
                      utd_ref[...], preferred_element_type=jnp.float32)
        h_td0 = _gates(gt0, ght[:B], hp0)

        m_dt = _smask([ht1[b] == t for b in range(B)])
        hp1 = jnp.where(m_dt > 0.0, h_td0, hp1s)

        @pl.when(_any([ht1[b] == t for b in range(B)]))
        def _corr_td():
            ctd_ref[...] = jnp.dot(_bf(hp1), utd_ref[...],
                                   preferred_element_type=jnp.float32)

        gh1t = jnp.where(m_dt > 0.0, ctd_ref[...], ght[B:])
        h_td1 = _gates(gt1, gh1t, hp1)
        out_ref[:, pl.ds(t, 1), H:2 * H] = h_td0.reshape(B, 1, H)
        out_ref[:, pl.ds(t + 1, 1), H:2 * H] = h_td1.reshape(B, 1, H)

        g2 = [htd_ref[pl.ds(ht2[b], 1), b, :] for b in range(B)]
        g3 = [htd_ref[pl.ds(ht3[b], 1), b, :] for b in range(B)]
        gath2 = jnp.concatenate(g2, axis=0)
        gath3 = jnp.concatenate(g3, axis=0)
        m2t0 = _smask([ht2[b] == t for b in range(B)])
        m2t1 = _smask([ht2[b] == t + 1 for b in range(B)])
        m3t0 = _smask([ht3[b] == t for b in range(B)])
        m3t1 = _smask([ht3[b] == t + 1 for b in range(B)])
        m3t2 = _smask([ht3[b] == t + 2 for b in range(B)])
        hp_n0 = jnp.where(m2t1 > 0.0, h_td1,
                          jnp.where(m2t0 > 0.0, h_td0, gath2))
        hp_n1 = jnp.where(m3t2 > 0.0, jnp.zeros_like(gath3),
                          jnp.where(m3t1 > 0.0, h_td1,
                                    jnp.where(m3t0 > 0.0, h_td0, gath3)))
        htd_ref[pl.ds(t, 2)] = jnp.concatenate(
            [h_td0.reshape(1, B, H), h_td1.reshape(1, B, H)], axis=0)

        return ((hs_n0, hs_n1), (hp_n0, hp_n1))

    def chunk(c, carry):
        base_dt = L - CH * (c + 1)
        xd = x_ref[pl.ds(base_dt, CH)].reshape(CH * B, D)
        gd = jnp.dot(xd, wdt_ref[...], preferred_element_type=jnp.float32)
        gxdt_ref[...] = (gd + bdt_ref[...]).reshape(CH, B, 3 * H)
        xt = x_ref[pl.ds(CH * c, CH)].reshape(CH * B, D)
        gt = jnp.dot(xt, wtd_ref[...], preferred_element_type=jnp.float32)
        gxtd_ref[...] = (gt + btd_ref[...]).reshape(CH, B, 3 * H)
        return jax.lax.fori_loop(
            0, CH // 2, lambda s2, cy: pair(c, s2, cy), carry)

    zero = jnp.zeros((B, H), jnp.float32)
    jax.lax.fori_loop(0, L // CH, chunk, ((zero, zero), (zero, zero)))


@functools.partial(jax.jit, static_argnames=())
def kernel(input, heads, W_dt, U_dt, b_dt, W_td, U_td, b_td):
    heads_i32 = heads.astype(jnp.int32)
    outputs = pl.pallas_call(
        _rnn_kernel,
        out_shape=jax.ShapeDtypeStruct((B, L, 2 * H), jnp.float32),
        in_specs=[
            pl.BlockSpec(memory_space=pltpu.SMEM),
            pl.BlockSpec(memory_space=pltpu.VMEM),
            pl.BlockSpec(memory_space=pltpu.VMEM),
            pl.BlockSpec(memory_space=pltpu.VMEM),
            pl.BlockSpec(memory_space=pltpu.VMEM),
            pl.BlockSpec(memory_space=pltpu.VMEM),
            pl.BlockSpec(memory_space=pltpu.VMEM),
            pl.BlockSpec(memory_space=pltpu.VMEM),
        ],
        out_specs=pl.BlockSpec(memory_space=pltpu.VMEM),
        scratch_shapes=[
            pltpu.VMEM((L, B, H), jnp.float32),
            pltpu.VMEM((L, B, H), jnp.float32),
            pltpu.VMEM((CH, B, 3 * H), jnp.float32),
            pltpu.VMEM((CH, B, 3 * H), jnp.float32),
            pltpu.VMEM((B, 3 * H), jnp.float32),
            pltpu.VMEM((B, 3 * H), jnp.float32),
        ],
    )(heads_i32, input, W_dt, U_dt.astype(jnp.bfloat16),
      b_dt.reshape(1, 3 * H), W_td, U_td.astype(jnp.bfloat16),
      b_td.reshape(1, 3 * H))
    output_t = outputs[:, 0, :][None, :, :]
    return outputs, output_t


# speedup vs baseline: 1.2272x; 1.2272x over previous
import functools

import jax
import jax.numpy as jnp
from jax.experimental import pallas as pl
from jax.experimental.pallas import tpu as pltpu

L, B, D, H = 1024, 8, 256, 256
CH = 64


def _gates(gx, gh, h_prev):
    r = jax.nn.sigmoid(gx[:, :H] + gh[:, :H])
    z = jax.nn.sigmoid(gx[:, H:2 * H] + gh[:, H:2 * H])
    n = jnp.tanh(gx[:, 2 * H:] + r * gh[:, 2 * H:])
    return (1.0 - z) * n + z * h_prev


def _smask(conds):
    cols = [jnp.broadcast_to(jnp.where(c, 1.0, 0.0), (1, 1)) for c in conds]
    return jnp.concatenate(cols, axis=0)


def _rnn_kernel(heads_ref, x_ref, wdt_ref, udt_ref, bdt_ref, wtd_ref,
                utd_ref, btd_ref, out_ref, child_sum_ref, htd_ref,
                gxdt_ref, gxtd_ref):
    child_sum_ref[...] = jnp.zeros((L, B, H), jnp.float32)

    def chunk(c, carry):
        base_dt = L - CH * (c + 1)
        xd = x_ref[pl.ds(base_dt, CH)].reshape(CH * B, D)
        gd = jnp.dot(xd, wdt_ref[...], preferred_element_type=jnp.float32)
        gxdt_ref[...] = (gd + bdt_ref[...]).reshape(CH, B, 3 * H)
        xt = x_ref[pl.ds(CH * c, CH)].reshape(CH * B, D)
        gt = jnp.dot(xt, wtd_ref[...], preferred_element_type=jnp.float32)
        gxtd_ref[...] = (gt + btd_ref[...]).reshape(CH, B, 3 * H)
        return jax.lax.fori_loop(0, CH, step_of(c), carry, unroll=2)

    def step_of(c):
        return lambda s, carry: step(c * CH + s, s, carry)

    def step(t, s, carry):
        h_sum, h_par = carry

        i = L - 1 - t
        gx = gxdt_ref[pl.ds(CH - 1 - s, 1)].reshape(B, 3 * H)
        gh = jnp.dot(h_sum.astype(jnp.bfloat16), udt_ref[...],
                     preferred_element_type=jnp.float32)
        h_dt = _gates(gx, gh, h_sum)
        out_ref[:, pl.ds(i, 1), 0:H] = h_dt.reshape(B, 1, H)

        inext = jnp.maximum(i - 1, 0)
        h_sum_next = child_sum_ref[pl.ds(inext, 1)].reshape(B, H)

        hbs = [heads_ref[b, i] for b in range(B)]
        u = _smask([hb == i - 1 for hb in hbs])
        g = jnp.where(i > 0, 1.0, 0.0)
        h_sum_next = h_sum_next + h_dt * (g * u)
        upd = h_dt * (g * (1.0 - u))
        loaded = [child_sum_ref[pl.ds(hbs[b], 1), b, :] for b in range(B)]
        for b in range(B):
            child_sum_ref[pl.ds(hbs[b], 1), b, :] = (
                loaded[b] + upd[b:b + 1, :])

        i2 = t
        gx2 = gxtd_ref[pl.ds(s, 1)].reshape(B, 3 * H)
        gh2 = jnp.dot(h_par.astype(jnp.bfloat16), utd_ref[...],
                      preferred_element_type=jnp.float32)
        h_td = _gates(gx2, gh2, h_par)
        out_ref[:, pl.ds(i2, 1), H:2 * H] = h_td.reshape(B, 1, H)

        tn = jnp.minimum(t + 1, L - 1)
        hb2s = [heads_ref[b, tn] for b in range(B)]
        rows = [htd_ref[pl.ds(hb2s[b], 1), b, :] for b in range(B)]
        gath = jnp.concatenate(rows, axis=0)
        u2 = _smask([hb2 == i2 for hb2 in hb2s])
        h_par_next = jnp.where(u2 > 0.0, h_td, gath)
        htd_ref[pl.ds(i2, 1)] = h_td.reshape(1, B, H)

        return (h_sum_next, h_par_next)

    zero = jnp.zeros((B, H), jnp.float32)
    jax.lax.fori_loop(0, L // CH, chunk, (zero, zero))


@functools.partial(jax.jit, static_argnames=())
def kernel(input, heads, W_dt, U_dt, b_dt, W_td, U_td, b_td):
    heads_i32 = heads.astype(jnp.int32)
    outputs = pl.pallas_call(
        _rnn_kernel,
        out_shape=jax.ShapeDtypeStruct((B, L, 2 * H), jnp.float32),
        in_specs=[
            pl.BlockSpec(memory_space=pltpu.SMEM),
            pl.BlockSpec(memory_space=pltpu.VMEM),
            pl.BlockSpec(memory_space=pltpu.VMEM),
            pl.BlockSpec(memory_space=pltpu.VMEM),
            pl.BlockSpec(memory_space=pltpu.VMEM),
            pl.BlockSpec(memory_space=pltpu.VMEM),
            pl.BlockSpec(memory_space=pltpu.VMEM),
            pl.BlockSpec(memory_space=pltpu.VMEM),
        ],
        out_specs=pl.BlockSpec(memory_space=pltpu.VMEM),
        scratch_shapes=[
            pltpu.VMEM((L, B, H), jnp.float32),
            pltpu.VMEM((L, B, H), jnp.float32),
            pltpu.VMEM((CH, B, 3 * H), jnp.float32),
            pltpu.VMEM((CH, B, 3 * H), jnp.float32),
        ],
    )(heads_i32, input, W_dt, U_dt.astype(jnp.bfloat16),
      b_dt.reshape(1, 3 * H), W_td, U_td.astype(jnp.bfloat16),
      b_td.reshape(1, 3 * H))
    output_t = outputs[:, 0, :][None, :, :]
    return outputs, output_t
